# baseline (speedup 1.0000x reference)
import jax
import jax.numpy as jnp
from jax import lax
from jax.experimental import pallas as pl
from jax.experimental.pallas import tpu as pltpu
from jax.experimental.pallas import tpu_sc as plsc

NC = 2
NS = 16
L = 16
NW = NC * NS
B = 16384
D = 16
GRP = 128 // D

NSTU = 1000000
NQST = 100000
FULL = 7812 * 128
SPAN = 244 * 128
CW = 1024
NWIN = 31
NSEL = B // L
PMASK = (1 << 14) - 1


def _irt_body(sid_hbm, qid_hbm, theta_hbm, ttail_hbm, alpha_hbm, beta_hbm,
              out_hbm,
              sfull, qfull, m_, c_, chv, aidx, alf, bef, pidx, pstg,
              sem, gsem, bsem, ssem):
    wid = lax.axis_index("s") * NC + lax.axis_index("c")
    lo = wid * SPAN
    span = jnp.where(wid == NW - 1, FULL - (NW - 1) * SPAN, SPAN)
    hi = jnp.where(wid == NW - 1, NSTU, lo + span)

    pltpu.sync_copy(sid_hbm, sfull)
    pltpu.sync_copy(qid_hbm, qfull)

    def sel_body(k, off):
        for u in range(4):
            r = (k * 4 + u) * L
            sv = sfull[pl.ds(r, L)]
            pos = r + lax.iota(jnp.int32, L)
            mask = (sv >= lo) & (sv < hi)
            packed = ((sv - lo) << 14) | pos
            plsc.store_compressed(m_.at[pl.ds(off, L)], packed, mask=mask)
            cnt = plsc.all_reduce_population_count(mask)
            off = off + cnt[0]
        return off

    nmatch = lax.fori_loop(0, NSEL // 4, sel_body, jnp.int32(0))

    iota = lax.iota(jnp.int32, L)

    def process(ws, we, strel, gidx, buf):
        def csel_body(t, coff):
            for u in range(4):
                r = (t * 4 + u) * L
                mv = m_[pl.ds(r, L)]
                locv = mv >> 14
                valid = (r + iota) < nmatch
                mask = valid & (locv >= ws) & (locv < we)
                plsc.store_compressed(c_.at[pl.ds(coff, L)], mv, mask=mask)
                cnt = plsc.all_reduce_population_count(mask)
                coff = coff + cnt[0]
            return coff

        ntrip = (nmatch + 4 * L - 1) // (4 * L)
        ccnt = lax.fori_loop(0, ntrip, csel_body, jnp.int32(0))

        @pl.when(ccnt > 0)
        def _():
            c0v = c_[pl.ds(0, L)]
            c_[pl.ds(ccnt, L)] = jnp.full((L,), c0v[0], jnp.int32)

        ngrp = (ccnt + L - 1) // L

        def fire_group(g, slot):
            mv = c_[pl.ds(g * L, L)]
            qv = plsc.load_gather(qfull, [mv & PMASK])
            a0 = aidx.at[slot]
            a0[0, pl.ds(0, L)] = qv >> 3
            a0[1, pl.ds(0, L)] = qv
            pltpu.async_copy(alpha_hbm.at[a0.at[0]], alf.at[slot], gsem)
            pltpu.async_copy(beta_hbm.at[a0.at[1]], bef.at[slot], bsem)

        def grp_body(g, gi):
            slot = g % 2

            @pl.when(g + 1 < ngrp)
            def _():
                fire_group(g + 1, (g + 1) % 2)

            pltpu.make_async_copy(
                alpha_hbm.at[aidx.at[0].at[0]], alf.at[0], gsem).wait()
            pltpu.make_async_copy(
                beta_hbm.at[aidx.at[0].at[1]], bef.at[0], bsem).wait()

            mv = c_[pl.ds(g * L, L)]
            colv = (mv >> 14) - strel
            qv = plsc.load_gather(qfull, [mv & PMASK])
            cba = (qv & 7) * D
            acc = bef.at[slot][pl.ds(0, L)]
            for d in range(D):
                t = plsc.load_gather(chv.at[buf],
                                     [jnp.full((L,), d, jnp.int32), colv])
                a = plsc.load_gather(alf.at[slot], [iota, cba + d])
                acc = acc + t * a
            pred = 1.0 / (1.0 + jnp.exp(-acc))

            sslot = gi % 4

            @pl.when(gi >= 4)
            def _():
                pltpu.make_async_copy(
                    pstg.at[0], out_hbm.at[pidx.at[0]], ssem).wait()

            pstg.at[sslot][pl.ds(0, L)] = pred
            pidx.at[sslot][pl.ds(0, L)] = mv & PMASK
            pltpu.async_copy(pstg.at[sslot], out_hbm.at[pidx.at[sslot]], ssem)
            return gi + 1

        @pl.when(ngrp > 0)
        def _():
            fire_group(0, 0)

        return lax.fori_loop(0, ngrp, grp_body, gidx)

    def win_start(ci):
        return pl.multiple_of(
            jnp.minimum(lo + ci * CW, FULL - CW).astype(jnp.int32), CW)

    def fire_win(ci):
        pltpu.async_copy(theta_hbm.at[:, pl.ds(win_start(ci), CW)],
                         chv.at[ci % 2], sem)

    fire_win(0)

    def win_body(ci, gidx):
        @pl.when(ci + 1 < NWIN)
        def _():
            fire_win(ci + 1)

        pltpu.make_async_copy(
            theta_hbm.at[:, pl.ds(0, CW)], chv.at[0], sem).wait()
        ws = ci * CW
        we = jnp.minimum(ci * CW + CW, span)
        return process(ws, we, win_start(ci) - lo, gidx, ci % 2)

    gidx = lax.fori_loop(0, NWIN, win_body, jnp.int32(0))

    pltpu.sync_copy(ttail_hbm, chv.at[0].at[:, pl.ds(0, 128)])
    gidx = process(jnp.int32(FULL) - lo, jnp.int32(NSTU) - lo,
                   jnp.int32(FULL) - lo, gidx, 0)

    def drain_body(i, carry):
        pltpu.make_async_copy(
            pstg.at[0], out_hbm.at[pidx.at[0]], ssem).wait()
        return carry

    lax.fori_loop(0, jnp.minimum(gidx, 4), drain_body, jnp.int32(0))


@jax.jit
def kernel(student_ids, question_ids, theta_w, alpha_w, beta_w):
    sid1 = student_ids.astype(jnp.int32)
    qid1 = question_ids.astype(jnp.int32)
    th_t = theta_w.T
    th_tail = jnp.pad(theta_w[FULL:, :].T, ((0, 0), (0, 128 - (NSTU - FULL))))
    al2 = alpha_w.reshape(NQST // GRP, 128)
    be1 = beta_w.reshape(-1)

    run = pl.kernel(
        _irt_body,
        out_type=jax.ShapeDtypeStruct((B,), jnp.float32),
        mesh=plsc.VectorSubcoreMesh(core_axis_name="c", subcore_axis_name="s"),
        scratch_types=[
            pltpu.VMEM((B,), jnp.int32),
            pltpu.VMEM((B,), jnp.int32),
            pltpu.VMEM((B + 4 * L,), jnp.int32),
            pltpu.VMEM((B + 4 * L,), jnp.int32),
            pltpu.VMEM((2, D, CW), jnp.float32),
            pltpu.VMEM((2, 2, L), jnp.int32),
            pltpu.VMEM((2, L, 128), jnp.float32),
            pltpu.VMEM((2, L), jnp.float32),
            pltpu.VMEM((4, L), jnp.int32),
            pltpu.VMEM((4, L), jnp.float32),
            pltpu.SemaphoreType.DMA,
            pltpu.SemaphoreType.DMA,
            pltpu.SemaphoreType.DMA,
            pltpu.SemaphoreType.DMA,
        ],
        compiler_params=pltpu.CompilerParams(
            needs_layout_passes=False, use_tc_tiling_on_sc=True),
    )
    out = run(sid1, qid1, th_t, th_tail, al2, be1)
    return out.reshape(B, 1)


# speedup vs baseline: 2.5137x; 2.5137x over previous
import jax
import jax.numpy as jnp
from jax import lax
from jax.experimental import pallas as pl
from jax.experimental.pallas import tpu as pltpu
from jax.experimental.pallas import tpu_sc as plsc

NC = 2
NS = 16
L = 16
NW = NC * NS
B = 16384
D = 16
GRP = 128 // D
BPW = B // NW
CH = 128
NCHUNK = BPW // CH

NSTU = 1000000
NQST = 100000
FULL = 7812 * 128
SPAN = 244 * 128
CW = 1024
NWIN = 31
NSEL = B // L


def _extract_body(sid_hbm, theta_hbm, tail_hbm, rows_hbm,
                  sfull, msid, mpos, cms, cmp_, chv, stage, sxw, sem, ssem):
    wid = lax.axis_index("s") * NC + lax.axis_index("c")
    lo = wid * SPAN
    span = jnp.where(wid == NW - 1, FULL - (NW - 1) * SPAN, SPAN)

    pltpu.sync_copy(sid_hbm, sfull)

    hi = jnp.where(wid == NW - 1, NSTU, lo + span)

    def sel_body(k, off):
        svs, poss, masks, cnts = [], [], [], []
        for u in range(4):
            r = (k * 4 + u) * L
            sv = sfull[pl.ds(r, L)]
            pos = r + lax.iota(jnp.int32, L)
            mask = (sv >= lo) & (sv < hi)
            svs.append(sv)
            poss.append(pos)
            masks.append(mask)
            cnts.append(plsc.all_reduce_population_count(mask)[0])
        offs = [off]
        for u in range(3):
            offs.append(offs[u] + cnts[u])
        for u in range(4):
            plsc.store_compressed(msid.at[pl.ds(offs[u], L)], svs[u],
                                  mask=masks[u])
            plsc.store_compressed(mpos.at[pl.ds(offs[u], L)], poss[u],
                                  mask=masks[u])
        return offs[3] + cnts[3]

    nmatch = lax.fori_loop(0, NSEL // 4, sel_body, jnp.int32(0))

    iota = lax.iota(jnp.int32, L)

    def process(ws, we, st, gidx, buf):
        def csel_body(t, coff):
            mss, mps, masks, cnts = [], [], [], []
            for u in range(4):
                r = (t * 4 + u) * L
                ms = msid[pl.ds(r, L)]
                mp = mpos[pl.ds(r, L)]
                valid = (r + iota) < nmatch
                mask = valid & (ms >= ws) & (ms < we)
                mss.append(ms - st)
                mps.append(mp)
                masks.append(mask)
                cnts.append(plsc.all_reduce_population_count(mask)[0])
            offs = [coff]
            for u in range(3):
                offs.append(offs[u] + cnts[u])
            for u in range(4):
                plsc.store_compressed(cms.at[pl.ds(offs[u], L)], mss[u],
                                      mask=masks[u])
                plsc.store_compressed(cmp_.at[pl.ds(offs[u], L)], mps[u],
                                      mask=masks[u])
            return offs[3] + cnts[3]

        ntrip = (nmatch + 4 * L - 1) // (4 * L)
        ccnt = lax.fori_loop(0, ntrip, csel_body, jnp.int32(0))

        @pl.when(ccnt > 0)
        def _():
            c0v = cms[pl.ds(0, L)]
            p0v = cmp_[pl.ds(0, L)]
            cms[pl.ds(ccnt, L)] = jnp.full((L,), c0v[0], jnp.int32)
            cmp_[pl.ds(ccnt, L)] = jnp.full((L,), p0v[0], jnp.int32)

        ngrp = (ccnt + L - 1) // L

        def grp_body(g, gi):
            cs_v = cms[pl.ds(g * L, L)]
            cp_v = cmp_[pl.ds(g * L, L)]
            sbuf = gi % 4

            @pl.when(gi >= 4)
            def _():
                pltpu.make_async_copy(
                    stage.at[0], rows_hbm.at[sxw.at[0]], ssem).wait()

            for j in range(L):
                feat = plsc.load_gather(
                    chv.at[buf], [iota, jnp.full((L,), cs_v[j], jnp.int32)])
                stage.at[sbuf][j, pl.ds(0, L)] = feat
            sxw.at[sbuf][pl.ds(0, L)] = cp_v
            pltpu.async_copy(stage.at[sbuf], rows_hbm.at[sxw.at[sbuf]], ssem)
            return gi + 1

        return lax.fori_loop(0, ngrp, grp_body, gidx)

    def win_start(ci):
        return pl.multiple_of(
            jnp.minimum(lo + ci * CW, FULL - CW).astype(jnp.int32), CW)

    def fire(ci):
        pltpu.async_copy(theta_hbm.at[:, pl.ds(win_start(ci), CW)],
                         chv.at[ci % 2], sem)

    fire(0)

    def win_body(ci, gidx):
        @pl.when(ci + 1 < NWIN)
        def _():
            fire(ci + 1)

        pltpu.make_async_copy(
            theta_hbm.at[:, pl.ds(0, CW)], chv.at[0], sem).wait()
        ws = lo + ci * CW
        we = lo + jnp.minimum(ci * CW + CW, span)
        return process(ws, we, win_start(ci), gidx, ci % 2)

    gidx = lax.fori_loop(0, NWIN, win_body, jnp.int32(0))

    pltpu.sync_copy(tail_hbm, chv.at[0].at[:, pl.ds(0, CH)])
    gidx = process(jnp.int32(FULL), jnp.int32(NSTU), jnp.int32(FULL),
                   gidx, 0)

    def drain_body(i, carry):
        pltpu.make_async_copy(
            stage.at[0], rows_hbm.at[sxw.at[0]], ssem).wait()
        return carry

    lax.fori_loop(0, jnp.minimum(gidx, 4), drain_body, jnp.int32(0))


def _score_body(qid_hbm, rows_hbm, alpha_hbm, beta_hbm, out_hbm,
                qraw, qhi, qlo, qidx, thv, alv, bev, outv, sem, bsem):
    wid = lax.axis_index("s") * NC + lax.axis_index("c")
    base = pl.multiple_of(wid * BPW, BPW)

    pltpu.sync_copy(qid_hbm.at[pl.ds(base, BPW)], qraw)

    for k in range(BPW // L):
        r = k * L
        qv = qraw[pl.ds(r, L)]
        qidx[r // CH, pl.ds(r % CH, L)] = qv
        qhi[r // CH, pl.ds(r % CH, L)] = qv >> 3
        qlo[pl.ds(r, L)] = (qv & 7) * D

    bcopies = [
        pltpu.async_copy(beta_hbm.at[qidx.at[j]],
                         bev.at[pl.ds(j * CH, CH)], bsem)
        for j in range(NCHUNK)
    ]

    def fire(c):
        buf = c % 2
        return [
            pltpu.async_copy(rows_hbm.at[pl.ds(base + c * CH, CH)],
                             thv.at[buf], sem),
            pltpu.async_copy(alpha_hbm.at[qhi.at[c]], alv.at[buf], sem),
        ]

    pend = fire(0)
    for c in bcopies:
        c.wait()

    iota = lax.iota(jnp.int32, L)
    for c in range(NCHUNK):
        buf = c % 2
        nxt = fire(c + 1) if c + 1 < NCHUNK else []
        for cp in pend:
            cp.wait()
        pend = nxt

        def blk_body(blk, carry):
            lr = blk * L
            r = c * CH + lr
            rows = lr + iota
            cba = qlo[pl.ds(r, L)]
            acc = bev[pl.ds(r, L)]
            for d in range(D):
                t = plsc.load_gather(thv.at[buf],
                                     [rows, jnp.full((L,), d, jnp.int32)])
                a = plsc.load_gather(alv.at[buf], [rows, cba + d])
                acc = acc + t * a
            outv[pl.ds(r, L)] = 1.0 / (1.0 + jnp.exp(-acc))
            return carry

        lax.fori_loop(0, CH // L, blk_body, jnp.int32(0))

    pltpu.sync_copy(outv, out_hbm.at[pl.ds(base, BPW)])


@jax.jit
def kernel(student_ids, question_ids, theta_w, alpha_w, beta_w):
    sid1 = student_ids.astype(jnp.int32)
    qid1 = question_ids.astype(jnp.int32)
    th_t = theta_w.T
    th_tail = jnp.pad(theta_w[FULL:, :].T,
                      ((0, 0), (0, CH - (NSTU - FULL))))
    al2 = alpha_w.reshape(NQST // GRP, 128)
    be1 = beta_w.reshape(-1)

    extract = pl.kernel(
        _extract_body,
        out_type=jax.ShapeDtypeStruct((B, 128), jnp.float32),
        mesh=plsc.VectorSubcoreMesh(core_axis_name="c", subcore_axis_name="s"),
        scratch_types=[
            pltpu.VMEM((B,), jnp.int32),
            pltpu.VMEM((B + 4 * L,), jnp.int32),
            pltpu.VMEM((B + 4 * L,), jnp.int32),
            pltpu.VMEM((B + 4 * L,), jnp.int32),
            pltpu.VMEM((B + 4 * L,), jnp.int32),
            pltpu.VMEM((2, D, CW), jnp.float32),
            pltpu.VMEM((4, L, 128), jnp.float32),
            pltpu.VMEM((4, L), jnp.int32),
            pltpu.SemaphoreType.DMA,
            pltpu.SemaphoreType.DMA,
        ],
        compiler_params=pltpu.CompilerParams(
            needs_layout_passes=False, use_tc_tiling_on_sc=True),
    )
    rows = extract(sid1, th_t, th_tail)

    score = pl.kernel(
        _score_body,
        out_type=jax.ShapeDtypeStruct((B,), jnp.float32),
        mesh=plsc.VectorSubcoreMesh(core_axis_name="c", subcore_axis_name="s"),
        scratch_types=[
            pltpu.VMEM((BPW,), jnp.int32),
            pltpu.VMEM((NCHUNK, CH), jnp.int32),
            pltpu.VMEM((BPW,), jnp.int32),
            pltpu.VMEM((NCHUNK, CH), jnp.int32),
            pltpu.VMEM((2, CH, 128), jnp.float32),
            pltpu.VMEM((2, CH, 128), jnp.float32),
            pltpu.VMEM((BPW,), jnp.float32),
            pltpu.VMEM((BPW,), jnp.float32),
            pltpu.SemaphoreType.DMA,
            pltpu.SemaphoreType.DMA,
        ],
        compiler_params=pltpu.CompilerParams(
            needs_layout_passes=False, use_tc_tiling_on_sc=True),
    )
    out = score(qid1, rows, al2, be1)
    return out.reshape(B, 1)


# speedup vs baseline: 2.6296x; 1.0461x over previous
import jax
import jax.numpy as jnp
from jax import lax
from jax.experimental import pallas as pl
from jax.experimental.pallas import tpu as pltpu
from jax.experimental.pallas import tpu_sc as plsc

NC = 2
NS = 16
L = 16
NW = NC * NS
B = 16384
D = 16
GRP = 128 // D
BPW = B // NW
CH = 128
NCHUNK = BPW // CH

NSTU = 1000000
NQST = 100000
FULL = 7812 * 128
SPAN = 244 * 128
CW = 2048
NWIN = 16
PMASK = (1 << 14) - 1
NSEL = B // L


def _extract_body(sid_hbm, theta_hbm, tail_hbm, rows_hbm,
                  sfull, m_, c_, chv, stage, sxw, sem, ssem):
    wid = lax.axis_index("s") * NC + lax.axis_index("c")
    lo = wid * SPAN
    span = jnp.where(wid == NW - 1, FULL - (NW - 1) * SPAN, SPAN)

    pltpu.sync_copy(sid_hbm, sfull)

    hi = jnp.where(wid == NW - 1, NSTU, lo + span)

    def sel_body(k, off):
        pks, masks, cnts = [], [], []
        for u in range(4):
            r = (k * 4 + u) * L
            sv = sfull[pl.ds(r, L)]
            pos = r + lax.iota(jnp.int32, L)
            mask = (sv >= lo) & (sv < hi)
            pks.append(((sv - lo) << 14) | pos)
            masks.append(mask)
            cnts.append(plsc.all_reduce_population_count(mask)[0])
        offs = [off]
        for u in range(3):
            offs.append(offs[u] + cnts[u])
        for u in range(4):
            plsc.store_compressed(m_.at[pl.ds(offs[u], L)], pks[u],
                                  mask=masks[u])
        return offs[3] + cnts[3]

    nmatch = lax.fori_loop(0, NSEL // 4, sel_body, jnp.int32(0))

    iota = lax.iota(jnp.int32, L)

    def process(wsp, wep, strel, gidx, buf):
        def csel_body(t, coff):
            pks, masks, cnts = [], [], []
            for u in range(4):
                r = (t * 4 + u) * L
                pk = m_[pl.ds(r, L)]
                valid = (r + iota) < nmatch
                mask = valid & (pk >= wsp) & (pk < wep)
                pks.append(pk)
                masks.append(mask)
                cnts.append(plsc.all_reduce_population_count(mask)[0])
            offs = [coff]
            for u in range(3):
                offs.append(offs[u] + cnts[u])
            for u in range(4):
                plsc.store_compressed(c_.at[pl.ds(offs[u], L)], pks[u],
                                      mask=masks[u])
            return offs[3] + cnts[3]

        ntrip = (nmatch + 4 * L - 1) // (4 * L)
        ccnt = lax.fori_loop(0, ntrip, csel_body, jnp.int32(0))

        @pl.when(ccnt > 0)
        def _():
            c0v = c_[pl.ds(0, L)]
            c_[pl.ds(ccnt, L)] = jnp.full((L,), c0v[0], jnp.int32)

        ngrp = (ccnt + L - 1) // L

        def grp_body(g, gi):
            pk_v = c_[pl.ds(g * L, L)]
            cs_v = (pk_v >> 14) - strel
            cp_v = pk_v & PMASK
            sbuf = gi % 4

            @pl.when(gi >= 4)
            def _():
                pltpu.make_async_copy(
                    stage.at[0], rows_hbm.at[sxw.at[0]], ssem).wait()

            for j in range(L):
                feat = plsc.load_gather(
                    chv.at[buf], [iota, jnp.full((L,), cs_v[j], jnp.int32)])
                stage.at[sbuf][j, pl.ds(0, L)] = feat
            sxw.at[sbuf][pl.ds(0, L)] = cp_v
            pltpu.async_copy(stage.at[sbuf], rows_hbm.at[sxw.at[sbuf]], ssem)
            return gi + 1

        return lax.fori_loop(0, ngrp, grp_body, gidx)

    def win_start(ci):
        return pl.multiple_of(
            jnp.minimum(lo + ci * CW, FULL - CW).astype(jnp.int32), CW)

    def fire(ci):
        pltpu.async_copy(theta_hbm.at[:, pl.ds(win_start(ci), CW)],
                         chv.at[ci % 2], sem)

    fire(0)

    def win_body(ci, gidx):
        @pl.when(ci + 1 < NWIN)
        def _():
            fire(ci + 1)

        pltpu.make_async_copy(
            theta_hbm.at[:, pl.ds(0, CW)], chv.at[0], sem).wait()
        wsp = (ci * CW) << 14
        wep = jnp.minimum(ci * CW + CW, span) << 14
        return process(wsp, wep, win_start(ci) - lo, gidx, ci % 2)

    gidx = lax.fori_loop(0, NWIN, win_body, jnp.int32(0))

    pltpu.sync_copy(tail_hbm, chv.at[0].at[:, pl.ds(0, CH)])
    gidx = process(jnp.minimum(jnp.int32(FULL) - lo, 1 << 15) << 14,
                   jnp.minimum(jnp.int32(NSTU) - lo, (1 << 15) + 1) << 14,
                   jnp.int32(FULL) - lo, gidx, 0)

    def drain_body(i, carry):
        pltpu.make_async_copy(
            stage.at[0], rows_hbm.at[sxw.at[0]], ssem).wait()
        return carry

    lax.fori_loop(0, jnp.minimum(gidx, 4), drain_body, jnp.int32(0))


def _score_body(qid_hbm, rows_hbm, alpha_hbm, beta_hbm, out_hbm,
                qraw, qhi, qlo, qidx, thv, alv, bev, outv, sem, bsem):
    wid = lax.axis_index("s") * NC + lax.axis_index("c")
    base = pl.multiple_of(wid * BPW, BPW)

    pltpu.sync_copy(qid_hbm.at[pl.ds(base, BPW)], qraw)

    for k in range(BPW // L):
        r = k * L
        qv = qraw[pl.ds(r, L)]
        qidx[r // CH, pl.ds(r % CH, L)] = qv
        qhi[r // CH, pl.ds(r % CH, L)] = qv >> 3
        qlo[pl.ds(r, L)] = (qv & 7) * D

    bcopies = [
        pltpu.async_copy(beta_hbm.at[qidx.at[j]],
                         bev.at[pl.ds(j * CH, CH)], bsem)
        for j in range(NCHUNK)
    ]

    def fire(c):
        buf = c % 2
        return [
            pltpu.async_copy(rows_hbm.at[pl.ds(base + c * CH, CH)],
                             thv.at[buf], sem),
            pltpu.async_copy(alpha_hbm.at[qhi.at[c]], alv.at[buf], sem),
        ]

    pend = fire(0)
    for c in bcopies:
        c.wait()

    iota = lax.iota(jnp.int32, L)
    for c in range(NCHUNK):
        buf = c % 2
        nxt = fire(c + 1) if c + 1 < NCHUNK else []
        for cp in pend:
            cp.wait()
        pend = nxt

        def blk_body(blk, carry):
            lr = blk * L
            r = c * CH + lr
            rows = lr + iota
            cba = qlo[pl.ds(r, L)]
            acc = bev[pl.ds(r, L)]
            for d in range(D):
                t = plsc.load_gather(thv.at[buf],
                                     [rows, jnp.full((L,), d, jnp.int32)])
                a = plsc.load_gather(alv.at[buf], [rows, cba + d])
                acc = acc + t * a
            outv[pl.ds(r, L)] = 1.0 / (1.0 + jnp.exp(-acc))
            return carry

        lax.fori_loop(0, CH // L, blk_body, jnp.int32(0))

    pltpu.sync_copy(outv, out_hbm.at[pl.ds(base, BPW)])


@jax.jit
def kernel(student_ids, question_ids, theta_w, alpha_w, beta_w):
    sid1 = student_ids.astype(jnp.int32)
    qid1 = question_ids.astype(jnp.int32)
    th_t = theta_w.T
    th_tail = jnp.pad(theta_w[FULL:, :].T,
                      ((0, 0), (0, CH - (NSTU - FULL))))
    al2 = alpha_w.reshape(NQST // GRP, 128)
    be1 = beta_w.reshape(-1)

    extract = pl.kernel(
        _extract_body,
        out_type=jax.ShapeDtypeStruct((B, 128), jnp.float32),
        mesh=plsc.VectorSubcoreMesh(core_axis_name="c", subcore_axis_name="s"),
        scratch_types=[
            pltpu.VMEM((B,), jnp.int32),
            pltpu.VMEM((B + 4 * L,), jnp.int32),
            pltpu.VMEM((B + 4 * L,), jnp.int32),
            pltpu.VMEM((2, D, CW), jnp.float32),
            pltpu.VMEM((4, L, 128), jnp.float32),
            pltpu.VMEM((4, L), jnp.int32),
            pltpu.SemaphoreType.DMA,
            pltpu.SemaphoreType.DMA,
        ],
        compiler_params=pltpu.CompilerParams(
            needs_layout_passes=False, use_tc_tiling_on_sc=True),
    )
    rows = extract(sid1, th_t, th_tail)

    score = pl.kernel(
        _score_body,
        out_type=jax.ShapeDtypeStruct((B,), jnp.float32),
        mesh=plsc.VectorSubcoreMesh(core_axis_name="c", subcore_axis_name="s"),
        scratch_types=[
            pltpu.VMEM((BPW,), jnp.int32),
            pltpu.VMEM((NCHUNK, CH), jnp.int32),
            pltpu.VMEM((BPW,), jnp.int32),
            pltpu.VMEM((NCHUNK, CH), jnp.int32),
            pltpu.VMEM((2, CH, 128), jnp.float32),
            pltpu.VMEM((2, CH, 128), jnp.float32),
            pltpu.VMEM((BPW,), jnp.float32),
            pltpu.VMEM((BPW,), jnp.float32),
            pltpu.SemaphoreType.DMA,
            pltpu.SemaphoreType.DMA,
        ],
        compiler_params=pltpu.CompilerParams(
            needs_layout_passes=False, use_tc_tiling_on_sc=True),
    )
    out = score(qid1, rows, al2, be1)
    return out.reshape(B, 1)
